# baseline (speedup 1.0000x reference)
import functools

import jax
import jax.numpy as jnp
from jax import lax
from jax.experimental import pallas as pl
from jax.experimental.pallas import tpu as pltpu
from jax.experimental.pallas import tpu_sc as plsc

_LANES = 16


def _interaction_table_kernel(cw_ref, swe_ref, swo_ref, cb_ref, sbe_ref,
                              sbo_ref, g_ref):
    dn = (((1,), (1,)), ((), ()))
    ge = lax.dot_general(cw_ref[...], swe_ref[...], dn,
                         preferred_element_type=jnp.float32,
                         precision=lax.Precision.HIGHEST)
    go = lax.dot_general(cw_ref[...], swo_ref[...], dn,
                         preferred_element_type=jnp.float32,
                         precision=lax.Precision.HIGHEST)
    ge = (ge + cb_ref[...] + sbe_ref[...]).astype(jnp.bfloat16)
    go = (go + cb_ref[...] + sbo_ref[...]).astype(jnp.bfloat16)
    lo = lax.bitcast_convert_type(ge, jnp.uint16).astype(jnp.uint32)
    hi = lax.bitcast_convert_type(go, jnp.uint16).astype(jnp.uint32)
    g_ref[...] = ((hi << 16) | lo).astype(jnp.int32)


def _build_interaction_table(c_weight, s_weight, c_biase, s_biase):
    v = c_weight.shape[0]
    h = v // 2
    sb = s_biase.reshape(v)
    return pl.pallas_call(
        _interaction_table_kernel,
        out_shape=jax.ShapeDtypeStruct((v, h), jnp.int32),
    )(c_weight, s_weight[0::2], s_weight[1::2], c_biase,
      sb[0::2].reshape(1, h), sb[1::2].reshape(1, h))


def _make_sc_gather(v, b, num_workers, chunk):
    per_w = b // num_workers
    rows = per_w // chunk
    h = v // 2
    mesh = plsc.VectorSubcoreMesh(core_axis_name="c", subcore_axis_name="s")

    @functools.partial(
        pl.kernel,
        mesh=mesh,
        out_type=jax.ShapeDtypeStruct((b,), jnp.float32),
        scratch_types=[
            pltpu.VMEM((per_w,), jnp.int32),
            pltpu.VMEM((per_w,), jnp.int32),
            pltpu.VMEM((per_w,), jnp.int32),
            pltpu.VMEM((per_w,), jnp.int32),
            pltpu.VMEM((per_w,), jnp.float32),
            pltpu.SemaphoreType.DMA,
        ],
    )
    def sc_gather(g_hbm, c_hbm, s_hbm, out_hbm, c_v, s_v, idx_v, pk_v,
                  val_v, sem):
        wid = lax.axis_index("s") * 2 + lax.axis_index("c")
        base = wid * per_w
        pltpu.sync_copy(c_hbm.at[pl.ds(base, per_w)], c_v)
        pltpu.sync_copy(s_hbm.at[pl.ds(base, per_w)], s_v)

        def idx_body(i, carry):
            sl = pl.ds(i * _LANES, _LANES)
            idx_v[sl] = c_v[sl] * h + lax.shift_right_logical(s_v[sl], 1)
            return carry

        lax.fori_loop(0, per_w // _LANES, idx_body, 0)
        copies = [
            pltpu.async_copy(
                g_hbm.at[idx_v.at[pl.ds(r * chunk, chunk)]],
                pk_v.at[pl.ds(r * chunk, chunk)],
                sem,
            )
            for r in range(rows)
        ]
        for cp in copies:
            cp.wait()

        def unpack_body(i, carry):
            sl = pl.ds(i * _LANES, _LANES)
            pk = pk_v[sl]
            odd = (s_v[sl] & 1) == 1
            bits = jnp.where(
                odd,
                pk & jnp.int32(-65536),
                lax.shift_left(pk, 16),
            )
            val_v[sl] = lax.bitcast_convert_type(bits, jnp.float32)
            return carry

        lax.fori_loop(0, per_w // _LANES, unpack_body, 0)
        pltpu.sync_copy(val_v, out_hbm.at[pl.ds(base, per_w)])

    return sc_gather


def kernel(c, s, c_weight, c_biase, s_weight, s_biase):
    v, _ = c_weight.shape
    b = c.shape[0]

    g = _build_interaction_table(c_weight, s_weight, c_biase, s_biase)
    g_flat = g.reshape(v * (v // 2))

    out = _make_sc_gather(v, b, 32, 128)(
        g_flat, c.astype(jnp.int32), s.astype(jnp.int32))
    return out.reshape(b, 1)


# speedup vs baseline: 1.2559x; 1.2559x over previous
import functools

import jax
import jax.numpy as jnp
from jax import lax
from jax.experimental import pallas as pl
from jax.experimental.pallas import tpu as pltpu
from jax.experimental.pallas import tpu_sc as plsc

_LANES = 16


def _interaction_table_kernel(cw_ref, sw_ref, cb_ref, sb_ref, g_ref):
    v = sw_ref.shape[0]
    h = v // 2
    dn = (((1,), (1,)), ((), ()))
    cw = cw_ref[...]
    ge = lax.dot_general(cw, sw_ref[0:h, :], dn,
                         preferred_element_type=jnp.float32,
                         precision=lax.Precision.HIGHEST)
    go = lax.dot_general(cw, sw_ref[h:v, :], dn,
                         preferred_element_type=jnp.float32,
                         precision=lax.Precision.HIGHEST)
    cb = cb_ref[...]
    sb = sb_ref[...]
    ge = (ge + cb + sb[:, 0:h]).astype(jnp.bfloat16)
    go = (go + cb + sb[:, h:v]).astype(jnp.bfloat16)
    lo = lax.bitcast_convert_type(ge, jnp.uint16).astype(jnp.uint32)
    hi = lax.bitcast_convert_type(go, jnp.uint16).astype(jnp.uint32)
    g_ref[...] = ((hi << 16) | lo).astype(jnp.int32)


def _build_interaction_table(c_weight, s_weight, c_biase, s_biase):
    v = c_weight.shape[0]
    return pl.pallas_call(
        _interaction_table_kernel,
        out_shape=jax.ShapeDtypeStruct((v, v // 2), jnp.int32),
    )(c_weight, s_weight, c_biase, s_biase.reshape(1, v))


def _make_sc_gather(v, b, num_workers, chunk):
    per_w = b // num_workers
    rows = per_w // chunk
    h = v // 2
    mesh = plsc.VectorSubcoreMesh(core_axis_name="c", subcore_axis_name="s")

    @functools.partial(
        pl.kernel,
        mesh=mesh,
        out_type=jax.ShapeDtypeStruct((b,), jnp.float32),
        scratch_types=[
            pltpu.VMEM((per_w,), jnp.int32),
            pltpu.VMEM((per_w,), jnp.int32),
            pltpu.VMEM((per_w,), jnp.int32),
            pltpu.VMEM((per_w,), jnp.int32),
            pltpu.VMEM((per_w,), jnp.float32),
            pltpu.SemaphoreType.DMA,
        ],
    )
    def sc_gather(g_hbm, c_hbm, s_hbm, out_hbm, c_v, s_v, idx_v, pk_v,
                  val_v, sem):
        wid = lax.axis_index("s") * 2 + lax.axis_index("c")
        base = wid * per_w
        pltpu.sync_copy(c_hbm.at[pl.ds(base, per_w)], c_v)
        pltpu.sync_copy(s_hbm.at[pl.ds(base, per_w)], s_v)

        def idx_body(i, carry):
            sl = pl.ds(i * _LANES, _LANES)
            sv = s_v[sl]
            idx_v[sl] = c_v[sl] * h + jnp.where(sv >= h, sv - h, sv)
            return carry

        lax.fori_loop(0, per_w // _LANES, idx_body, 0)
        copies = [
            pltpu.async_copy(
                g_hbm.at[idx_v.at[pl.ds(r * chunk, chunk)]],
                pk_v.at[pl.ds(r * chunk, chunk)],
                sem,
            )
            for r in range(rows)
        ]
        for cp in copies:
            cp.wait()

        def unpack_body(i, carry):
            sl = pl.ds(i * _LANES, _LANES)
            pk = pk_v[sl]
            bits = jnp.where(
                s_v[sl] >= h,
                pk & jnp.int32(-65536),
                lax.shift_left(pk, 16),
            )
            val_v[sl] = lax.bitcast_convert_type(bits, jnp.float32)
            return carry

        lax.fori_loop(0, per_w // _LANES, unpack_body, 0)
        pltpu.sync_copy(val_v, out_hbm.at[pl.ds(base, per_w)])

    return sc_gather


def kernel(c, s, c_weight, c_biase, s_weight, s_biase):
    v, _ = c_weight.shape
    b = c.shape[0]

    g = _build_interaction_table(c_weight, s_weight, c_biase, s_biase)
    g_flat = g.reshape(v * (v // 2))

    out = _make_sc_gather(v, b, 32, 128)(
        g_flat, c.astype(jnp.int32), s.astype(jnp.int32))
    return out.reshape(b, 1)
